# Initial kernel scaffold; baseline (speedup 1.0000x reference)
import functools

import jax
import jax.numpy as jnp
from jax import lax
from jax.experimental import pallas as pl
from jax.experimental.pallas import tpu as pltpu
from jax.experimental.pallas import tpu_sc as plsc

NUM_BINS = 255
MIN_V = -20.0
MAX_V = 20.0
BIN_WIDTH = (MAX_V - MIN_V) / (NUM_BINS - 1)

N = 262144
NC = 2
NS = 16
NW = NC * NS
RW = N // NW
C = 128
NCH = RW // C
CW = C * NUM_BINS
L = 16


def _sc_body(values_hbm, out_hbm, vals_v, buf0, buf1, idx0, idx1, sem0, sem1):
    wid = lax.axis_index("s") * NC + lax.axis_index("c")
    row0 = wid * RW

    pltpu.sync_copy(values_hbm.at[pl.ds(row0, RW)], vals_v)

    bufs = (buf0, buf1)
    idxs = (idx0, idx1)
    sems = (sem0, sem1)

    zeros = jnp.zeros((L,), jnp.float32)
    lane = lax.iota(jnp.int32, L)

    def memset_body(i, carry):
        buf0[pl.ds(i * L, L)] = zeros
        buf1[pl.ds(i * L, L)] = zeros
        return carry

    lax.fori_loop(0, CW // L, memset_body, 0, unroll=8)

    def process(chunk, b):
        buf = bufs[b]
        idx = idxs[b]
        vbase = chunk * C
        for g in range(C // L):
            v = vals_v[pl.ds(vbase + g * L, L)]
            v = jnp.minimum(jnp.maximum(v, MIN_V), MAX_V)
            norm = (v - MIN_V) / BIN_WIDTH
            lo = norm.astype(jnp.int32)
            lo = jnp.minimum(lo, NUM_BINS - 1)
            lof = lo.astype(jnp.float32)
            up = jnp.where(norm > lof, lo + 1, lo)
            up = jnp.minimum(up, NUM_BINS - 1)
            uw = norm - lof
            lw = 1.0 - uw
            rbase = g * L * NUM_BINS
            flat_l = lane * NUM_BINS + rbase + lo
            flat_u = lane * NUM_BINS + rbase + up
            plsc.store_scatter(buf, [flat_l], lw)
            plsc.addupdate_scatter(buf, [flat_u], uw)
            idx[pl.ds(g * L, L)] = flat_l
            idx[pl.ds(C + g * L, L)] = flat_u

    def issue(chunk, b):
        dst = out_hbm.at[pl.ds((row0 + chunk * C) * NUM_BINS, CW)]
        pltpu.async_copy(bufs[b], dst, sems[b])

    def drain(chunk, b):
        dst = out_hbm.at[pl.ds((row0 + chunk * C) * NUM_BINS, CW)]
        pltpu.make_async_copy(bufs[b], dst, sems[b]).wait()

    for b in range(2):
        process(b, b)
        issue(b, b)

    def pair_body(p, carry):
        for b in range(2):
            chunk = p * 2 + b
            drain(chunk - 2, b)
            for g in range(C // L):
                plsc.store_scatter(bufs[b], [idxs[b][pl.ds(g * L, L)]], zeros)
                plsc.store_scatter(bufs[b], [idxs[b][pl.ds(C + g * L, L)]], zeros)
            process(chunk, b)
            issue(chunk, b)
        return carry

    lax.fori_loop(1, NCH // 2, pair_body, 0)

    for b in range(2):
        drain(NCH - 2 + b, b)


@functools.partial(
    pl.kernel,
    out_type=jax.ShapeDtypeStruct((N * NUM_BINS,), jnp.float32),
    mesh=plsc.VectorSubcoreMesh(core_axis_name="c", subcore_axis_name="s"),
    scratch_types=[
        pltpu.VMEM((RW,), jnp.float32),
        pltpu.VMEM((CW,), jnp.float32),
        pltpu.VMEM((CW,), jnp.float32),
        pltpu.VMEM((2 * C,), jnp.int32),
        pltpu.VMEM((2 * C,), jnp.int32),
        pltpu.SemaphoreType.DMA,
        pltpu.SemaphoreType.DMA,
    ],
)
def _two_hot_sc(values_hbm, out_hbm, vals_v, buf0, buf1, idx0, idx1, sem0, sem1):
    _sc_body(values_hbm, out_hbm, vals_v, buf0, buf1, idx0, idx1, sem0, sem1)


def kernel(values):
    flat = _two_hot_sc(values)
    return flat.reshape(N, NUM_BINS)


# speedup vs baseline: 4.0654x; 4.0654x over previous
import functools

import jax
import jax.numpy as jnp
from jax import lax
from jax.experimental import pallas as pl
from jax.experimental.pallas import tpu as pltpu
from jax.experimental.pallas import tpu_sc as plsc

NUM_BINS = 255
MIN_V = -20.0
MAX_V = 20.0
BIN_WIDTH = (MAX_V - MIN_V) / (NUM_BINS - 1)

N = 262144
NC = 2
NS = 16
NW = NC * NS
RW = N // NW
C = 128
NCH = RW // C
CW = C * NUM_BINS
L = 16


def _sc_body(values_hbm, out_hbm, vals_v, buf0, buf1, idx0, idx1, sem0, sem1):
    wid = lax.axis_index("s") * NC + lax.axis_index("c")
    row0 = wid * RW

    pltpu.sync_copy(values_hbm.at[pl.ds(row0, RW)], vals_v)

    bufs = (buf0, buf1)
    idxs = (idx0, idx1)
    sems = (sem0, sem1)

    zeros = jnp.zeros((L,), jnp.float32)
    lane = lax.iota(jnp.int32, L)

    def memset_body(i, carry):
        buf0[pl.ds(i * L, L)] = zeros
        buf1[pl.ds(i * L, L)] = zeros
        return carry

    lax.fori_loop(0, CW // L, memset_body, 0, unroll=8)

    def process(chunk, b):
        buf = bufs[b]
        idx = idxs[b]
        vbase = chunk * C
        for g in range(C // L):
            v = vals_v[pl.ds(vbase + g * L, L)]
            v = jnp.minimum(jnp.maximum(v, MIN_V), MAX_V)
            norm = (v - MIN_V) / BIN_WIDTH
            lo = norm.astype(jnp.int32)
            lo = jnp.minimum(lo, NUM_BINS - 1)
            lof = lo.astype(jnp.float32)
            up = jnp.where(norm > lof, lo + 1, lo)
            up = jnp.minimum(up, NUM_BINS - 1)
            uw = norm - lof
            lw = 1.0 - uw
            rbase = g * L * NUM_BINS
            flat_l = lane * NUM_BINS + rbase + lo
            flat_u = lane * NUM_BINS + rbase + up
            plsc.store_scatter(buf, [flat_l], lw)
            plsc.addupdate_scatter(buf, [flat_u], uw)
            idx[pl.ds(g * L, L)] = flat_l
            idx[pl.ds(C + g * L, L)] = flat_u

    def issue(chunk, b):
        dst = out_hbm.at[pl.ds((row0 + chunk * C) * NUM_BINS, CW)]
        pltpu.async_copy(bufs[b], dst, sems[b])

    def drain(chunk, b):
        dst = out_hbm.at[pl.ds((row0 + chunk * C) * NUM_BINS, CW)]
        pltpu.make_async_copy(bufs[b], dst, sems[b]).wait()

    for b in range(2):
        process(b, b)
        issue(b, b)

    def pair_body(p, carry):
        for b in range(2):
            chunk = p * 2 + b
            drain(chunk - 2, b)
            for g in range(C // L):
                plsc.store_scatter(bufs[b], [idxs[b][pl.ds(g * L, L)]], zeros)
                plsc.store_scatter(bufs[b], [idxs[b][pl.ds(C + g * L, L)]], zeros)
            process(chunk, b)
            issue(chunk, b)
        return carry

    lax.fori_loop(1, NCH // 2, pair_body, 0)

    for b in range(2):
        drain(NCH - 2 + b, b)


@functools.partial(
    pl.kernel,
    out_type=jax.ShapeDtypeStruct((N * NUM_BINS,), jnp.float32),
    mesh=plsc.VectorSubcoreMesh(core_axis_name="c", subcore_axis_name="s"),
    compiler_params=pltpu.CompilerParams(needs_layout_passes=False),
    scratch_types=[
        pltpu.VMEM((RW,), jnp.float32),
        pltpu.VMEM((CW,), jnp.float32),
        pltpu.VMEM((CW,), jnp.float32),
        pltpu.VMEM((2 * C,), jnp.int32),
        pltpu.VMEM((2 * C,), jnp.int32),
        pltpu.SemaphoreType.DMA,
        pltpu.SemaphoreType.DMA,
    ],
)
def _two_hot_sc(values_hbm, out_hbm, vals_v, buf0, buf1, idx0, idx1, sem0, sem1):
    _sc_body(values_hbm, out_hbm, vals_v, buf0, buf1, idx0, idx1, sem0, sem1)


def kernel(values):
    flat = _two_hot_sc(values)
    return flat.reshape(N, NUM_BINS)


# speedup vs baseline: 19.5877x; 4.8182x over previous
import functools

import jax
import jax.numpy as jnp
from jax import lax
from jax.experimental import pallas as pl
from jax.experimental.pallas import tpu as pltpu
from jax.experimental.pallas import tpu_sc as plsc

NUM_BINS = 255
MIN_V = -20.0
MAX_V = 20.0
BIN_WIDTH = (MAX_V - MIN_V) / (NUM_BINS - 1)

N = 262144
NC = 2
NS = 16
NW = NC * NS
RW = N // NW
C = 128
NCH = RW // C
L = 16


def _sc_body(values_hbm, out_hbm, vals_v, buf0, buf1, idx0, idx1, sem0, sem1):
    wid = lax.axis_index("s") * NC + lax.axis_index("c")
    row0 = wid * RW

    pltpu.sync_copy(values_hbm.at[pl.ds(row0, RW)], vals_v)

    bufs = (buf0, buf1)
    idxs = (idx0, idx1)
    sems = (sem0, sem1)

    zeros = jnp.zeros((L,), jnp.float32)
    lane = lax.iota(jnp.int32, L)

    def memset_rows(buf):
        def body(r, carry):
            for g in range(NUM_BINS // L):
                buf[r, pl.ds(g * L, L)] = zeros
            buf[r, pl.ds(NUM_BINS - L, L)] = zeros
            return carry
        lax.fori_loop(0, C, body, 0)

    memset_rows(buf0)
    memset_rows(buf1)

    def process(chunk, b):
        buf = bufs[b]
        idx = idxs[b]
        vbase = chunk * C
        for g in range(C // L):
            v = vals_v[pl.ds(vbase + g * L, L)]
            v = jnp.minimum(jnp.maximum(v, MIN_V), MAX_V)
            norm = (v - MIN_V) / BIN_WIDTH
            lo = norm.astype(jnp.int32)
            lo = jnp.minimum(lo, NUM_BINS - 1)
            lof = lo.astype(jnp.float32)
            up = jnp.where(norm > lof, lo + 1, lo)
            up = jnp.minimum(up, NUM_BINS - 1)
            uw = norm - lof
            lw = 1.0 - uw
            rows = lane + (g * L)
            plsc.store_scatter(buf, [rows, lo], lw)
            plsc.addupdate_scatter(buf, [rows, up], uw)
            idx[pl.ds(g * L, L)] = lo
            idx[pl.ds(C + g * L, L)] = up

    def issue(chunk, b):
        dst = out_hbm.at[pl.ds(row0 + chunk * C, C)]
        pltpu.async_copy(bufs[b], dst, sems[b])

    def drain(chunk, b):
        dst = out_hbm.at[pl.ds(row0 + chunk * C, C)]
        pltpu.make_async_copy(bufs[b], dst, sems[b]).wait()

    for b in range(2):
        process(b, b)
        issue(b, b)

    def pair_body(p, carry):
        for b in range(2):
            chunk = p * 2 + b
            drain(chunk - 2, b)
            for g in range(C // L):
                rows = lane + (g * L)
                plsc.store_scatter(bufs[b], [rows, idxs[b][pl.ds(g * L, L)]], zeros)
                plsc.store_scatter(bufs[b], [rows, idxs[b][pl.ds(C + g * L, L)]], zeros)
            process(chunk, b)
            issue(chunk, b)
        return carry

    lax.fori_loop(1, NCH // 2, pair_body, 0)

    for b in range(2):
        drain(NCH - 2 + b, b)


@functools.partial(
    pl.kernel,
    out_type=jax.ShapeDtypeStruct((N, NUM_BINS), jnp.float32),
    mesh=plsc.VectorSubcoreMesh(core_axis_name="c", subcore_axis_name="s"),
    compiler_params=pltpu.CompilerParams(needs_layout_passes=False),
    scratch_types=[
        pltpu.VMEM((RW,), jnp.float32),
        pltpu.VMEM((C, NUM_BINS), jnp.float32),
        pltpu.VMEM((C, NUM_BINS), jnp.float32),
        pltpu.VMEM((2 * C,), jnp.int32),
        pltpu.VMEM((2 * C,), jnp.int32),
        pltpu.SemaphoreType.DMA,
        pltpu.SemaphoreType.DMA,
    ],
)
def _two_hot_sc(values_hbm, out_hbm, vals_v, buf0, buf1, idx0, idx1, sem0, sem1):
    _sc_body(values_hbm, out_hbm, vals_v, buf0, buf1, idx0, idx1, sem0, sem1)


def kernel(values):
    return _two_hot_sc(values)
